# Initial kernel scaffold; baseline (speedup 1.0000x reference)
import jax
import jax.numpy as jnp
from jax.experimental import pallas as pl


def kernel(x, edge_index, W, b):
    raise NotImplementedError("write your pallas kernel here")


# speedup vs baseline: 31.1453x; 31.1453x over previous
import jax
import jax.numpy as jnp
from jax import lax
from jax.experimental import pallas as pl
from jax.experimental.pallas import tpu as pltpu
from jax.experimental.pallas import tpu_sc as plsc

N = 10000
E = 320000
D = 128
C = 16
NC = 2
NS = 16
NW = NC * NS
CHUNK = 128
EPW = -(-(E // NW) // CHUNK) * CHUNK
NCH = EPW // CHUNK
AGG_ROWS = 10240
RPS = AGG_ROWS // NS


_mesh = plsc.VectorSubcoreMesh(core_axis_name="c", subcore_axis_name="s")
_sc_params = pltpu.CompilerParams(use_tc_tiling_on_sc=False)


def _hop_body(tab, src_r, dst_r, zero128, out, src_v, dst_v, rows_v, sem, agg):
    c = lax.axis_index("c")
    s = lax.axis_index("s")
    wid = c * NS + s
    pltpu.sync_copy(zero128, rows_v)
    for k in range(RPS // CHUNK):
        pltpu.sync_copy(rows_v, agg.at[pl.ds(s * RPS + k * CHUNK, CHUNK)])
    plsc.subcore_barrier()
    pltpu.sync_copy(src_r.at[wid], src_v)
    pltpu.sync_copy(dst_r.at[wid], dst_v)

    def chunk(j, carry):
        pltpu.async_copy(tab.at[src_v.at[j]], rows_v, sem).wait()
        pltpu.sync_copy(rows_v, agg.at[dst_v.at[j]], add=True)
        return carry

    lax.fori_loop(0, NCH, chunk, 0)
    plsc.subcore_barrier()
    pltpu.sync_copy(agg.at[pl.ds(s * RPS, RPS)],
                    out.at[pl.ds(c * AGG_ROWS + s * RPS, RPS)])


_hop = pl.kernel(
    _hop_body,
    out_type=jax.ShapeDtypeStruct((NC * AGG_ROWS, C), jnp.float32),
    mesh=_mesh,
    scratch_types=[
        pltpu.VMEM((NCH, CHUNK), jnp.int32),
        pltpu.VMEM((NCH, CHUNK), jnp.int32),
        pltpu.VMEM((CHUNK, C), jnp.float32),
        pltpu.SemaphoreType.DMA,
        pltpu.VMEM_SHARED((AGG_ROWS, C), jnp.float32),
    ],
    compiler_params=_sc_params,
)


def _deg_body(dst_r, zero128, one128, out, dst_v, rows_v, agg):
    c = lax.axis_index("c")
    s = lax.axis_index("s")
    wid = c * NS + s
    pltpu.sync_copy(zero128, rows_v)
    for k in range(RPS // CHUNK):
        pltpu.sync_copy(rows_v, agg.at[pl.ds(s * RPS + k * CHUNK, CHUNK)])
    plsc.subcore_barrier()
    pltpu.sync_copy(one128, rows_v)
    pltpu.sync_copy(dst_r.at[wid], dst_v)

    def chunk(j, carry):
        pltpu.sync_copy(rows_v, agg.at[dst_v.at[j]], add=True)
        return carry

    lax.fori_loop(0, NCH, chunk, 0)
    plsc.subcore_barrier()
    pltpu.sync_copy(agg.at[pl.ds(s * RPS, RPS)],
                    out.at[pl.ds(c * AGG_ROWS + s * RPS, RPS)])


_deg = pl.kernel(
    _deg_body,
    out_type=jax.ShapeDtypeStruct((NC * AGG_ROWS, C), jnp.float32),
    mesh=_mesh,
    scratch_types=[
        pltpu.VMEM((NCH, CHUNK), jnp.int32),
        pltpu.VMEM((CHUNK, C), jnp.float32),
        pltpu.VMEM_SHARED((AGG_ROWS, C), jnp.float32),
    ],
    compiler_params=_sc_params,
)


def _linear_body(x_ref, w_ref, o_ref):
    o_ref[...] = lax.dot_general(
        x_ref[...], w_ref[...], (((1,), (1,)), ((), ())),
        preferred_element_type=jnp.float32,
        precision=lax.Precision.HIGHEST,
    )


def _prep_body(da_ref, db_ref, y_ref, dinv_ref, z0_ref):
    dinv = lax.rsqrt(da_ref[...] + db_ref[...] + 1.0)
    dinv_ref[...] = dinv
    z0_ref[...] = dinv * y_ref[...]


def _mid_body(aa_ref, ab_ref, z0_ref, dinv_ref, z1_ref):
    dinv = dinv_ref[...]
    z1_ref[...] = dinv * dinv * (aa_ref[...] + ab_ref[...] + z0_ref[...])


def _final_body(aa_ref, ab_ref, z1_ref, dinv_ref, b_ref, o_ref):
    h2 = dinv_ref[...] * (aa_ref[...] + ab_ref[...] + z1_ref[...])
    logits = h2 + b_ref[...]
    m = jnp.max(logits, axis=1, keepdims=True)
    lse = m + jnp.log(jnp.sum(jnp.exp(logits - m), axis=1, keepdims=True))
    o_ref[...] = logits - lse


def _tc(body, n_out, *ins):
    outs = tuple(jax.ShapeDtypeStruct((N, C), jnp.float32) for _ in range(n_out))
    return pl.pallas_call(body, out_shape=outs if n_out > 1 else outs[0])(*ins)


def kernel(x, edge_index, W, b):
    src, dst = edge_index[0], edge_index[1]
    pad = NW * EPW - E
    if pad:
        src = jnp.concatenate([src, jnp.zeros((pad,), src.dtype)])
        dst = jnp.concatenate([dst, jnp.full((pad,), N, dst.dtype)])
    src_r = src.reshape(NW, NCH, CHUNK)
    dst_r = dst.reshape(NW, NCH, CHUNK)
    zero128 = jnp.zeros((CHUNK, C), jnp.float32)
    one128 = jnp.ones((CHUNK, C), jnp.float32)

    y = _tc(_linear_body, 1, x, W)
    degp = _deg(dst_r, zero128, one128)
    dinv, z0 = _tc(_prep_body, 2, degp[:N], degp[AGG_ROWS:AGG_ROWS + N], y)
    a1 = _hop(z0, src_r, dst_r, zero128)
    z1 = _tc(_mid_body, 1, a1[:N], a1[AGG_ROWS:AGG_ROWS + N], z0, dinv)
    a2 = _hop(z1, src_r, dst_r, zero128)
    return _tc(_final_body, 1, a2[:N], a2[AGG_ROWS:AGG_ROWS + N], z1, dinv,
               b.reshape(1, C))


# speedup vs baseline: 41.9833x; 1.3480x over previous
import jax
import jax.numpy as jnp
from jax import lax
from jax.experimental import pallas as pl
from jax.experimental.pallas import tpu as pltpu
from jax.experimental.pallas import tpu_sc as plsc

N = 10000
E = 320000
D = 128
C = 16
NC = 2
NS = 16
NW = NC * NS
CHUNK = 128
EPW = -(-(E // NW) // CHUNK) * CHUNK
NCH = EPW // CHUNK
AGG_ROWS = 10240
RPS = AGG_ROWS // NS


_mesh = plsc.VectorSubcoreMesh(core_axis_name="c", subcore_axis_name="s")
_sc_params = pltpu.CompilerParams(use_tc_tiling_on_sc=False)


NBUF = 8
NG = 3


def _hop_body(tab, src_r, dst_r, zero128, out, src_v, dst_v, rows_v, gsem,
              ssem, agg):
    c = lax.axis_index("c")
    s = lax.axis_index("s")
    wid = c * NS + s
    pltpu.sync_copy(zero128, rows_v.at[0])
    for k in range(RPS // CHUNK):
        pltpu.sync_copy(rows_v.at[0], agg.at[pl.ds(s * RPS + k * CHUNK, CHUNK)])
    plsc.subcore_barrier()
    pltpu.sync_copy(src_r.at[wid], src_v)
    pltpu.sync_copy(dst_r.at[wid], dst_v)

    for j in range(NG):
        pltpu.async_copy(tab.at[src_v.at[j]], rows_v.at[j], gsem)

    def chunk(j, carry):
        @pl.when(j >= NBUF - NG)
        def _():
            pltpu.make_async_copy(zero128, rows_v.at[0], ssem).wait()

        @pl.when(j + NG < NCH)
        def _():
            pltpu.async_copy(tab.at[src_v.at[lax.rem(j + NG, NCH)]],
                             rows_v.at[lax.rem(j + NG, NBUF)], gsem)

        pltpu.make_async_copy(zero128, rows_v.at[0], gsem).wait()
        pltpu.async_copy(rows_v.at[lax.rem(j, NBUF)], agg.at[dst_v.at[j]],
                         ssem, add=True)
        return carry

    lax.fori_loop(0, NCH, chunk, 0)

    def drain(j, carry):
        pltpu.make_async_copy(zero128, rows_v.at[0], ssem).wait()
        return carry

    lax.fori_loop(0, NBUF - NG, drain, 0)
    plsc.subcore_barrier()
    pltpu.sync_copy(agg.at[pl.ds(s * RPS, RPS)],
                    out.at[pl.ds(c * AGG_ROWS + s * RPS, RPS)])


_hop = pl.kernel(
    _hop_body,
    out_type=jax.ShapeDtypeStruct((NC * AGG_ROWS, C), jnp.float32),
    mesh=_mesh,
    scratch_types=[
        pltpu.VMEM((NCH, CHUNK), jnp.int32),
        pltpu.VMEM((NCH, CHUNK), jnp.int32),
        pltpu.VMEM((NBUF, CHUNK, C), jnp.float32),
        pltpu.SemaphoreType.DMA,
        pltpu.SemaphoreType.DMA,
        pltpu.VMEM_SHARED((AGG_ROWS, C), jnp.float32),
    ],
    compiler_params=_sc_params,
)


def _deg_body(dst_r, zero128, one128, out, dst_v, rows_v, ssem, agg):
    c = lax.axis_index("c")
    s = lax.axis_index("s")
    wid = c * NS + s
    pltpu.sync_copy(zero128, rows_v)
    for k in range(RPS // CHUNK):
        pltpu.sync_copy(rows_v, agg.at[pl.ds(s * RPS + k * CHUNK, CHUNK)])
    plsc.subcore_barrier()
    pltpu.sync_copy(one128, rows_v)
    pltpu.sync_copy(dst_r.at[wid], dst_v)

    def chunk(j, carry):
        pltpu.async_copy(rows_v, agg.at[dst_v.at[j]], ssem, add=True)

        @pl.when(j >= 8)
        def _():
            pltpu.make_async_copy(zero128, rows_v, ssem).wait()

        return carry

    lax.fori_loop(0, NCH, chunk, 0)

    def drain(j, carry):
        pltpu.make_async_copy(zero128, rows_v, ssem).wait()
        return carry

    lax.fori_loop(0, min(NCH, 8), drain, 0)
    plsc.subcore_barrier()
    pltpu.sync_copy(agg.at[pl.ds(s * RPS, RPS)],
                    out.at[pl.ds(c * AGG_ROWS + s * RPS, RPS)])


_deg = pl.kernel(
    _deg_body,
    out_type=jax.ShapeDtypeStruct((NC * AGG_ROWS, C), jnp.float32),
    mesh=_mesh,
    scratch_types=[
        pltpu.VMEM((NCH, CHUNK), jnp.int32),
        pltpu.VMEM((CHUNK, C), jnp.float32),
        pltpu.SemaphoreType.DMA,
        pltpu.VMEM_SHARED((AGG_ROWS, C), jnp.float32),
    ],
    compiler_params=_sc_params,
)


def _linear_body(x_ref, w_ref, o_ref):
    o_ref[...] = lax.dot_general(
        x_ref[...], w_ref[...], (((1,), (1,)), ((), ())),
        preferred_element_type=jnp.float32,
        precision=lax.Precision.HIGHEST,
    )


def _prep_body(da_ref, db_ref, y_ref, dinv_ref, z0_ref):
    dinv = lax.rsqrt(da_ref[...] + db_ref[...] + 1.0)
    dinv_ref[...] = dinv
    z0_ref[...] = dinv * y_ref[...]


def _mid_body(aa_ref, ab_ref, z0_ref, dinv_ref, z1_ref):
    dinv = dinv_ref[...]
    z1_ref[...] = dinv * dinv * (aa_ref[...] + ab_ref[...] + z0_ref[...])


def _final_body(aa_ref, ab_ref, z1_ref, dinv_ref, b_ref, o_ref):
    h2 = dinv_ref[...] * (aa_ref[...] + ab_ref[...] + z1_ref[...])
    logits = h2 + b_ref[...]
    m = jnp.max(logits, axis=1, keepdims=True)
    lse = m + jnp.log(jnp.sum(jnp.exp(logits - m), axis=1, keepdims=True))
    o_ref[...] = logits - lse


def _tc(body, n_out, *ins):
    outs = tuple(jax.ShapeDtypeStruct((N, C), jnp.float32) for _ in range(n_out))
    return pl.pallas_call(body, out_shape=outs if n_out > 1 else outs[0])(*ins)


def kernel(x, edge_index, W, b):
    src, dst = edge_index[0], edge_index[1]
    pad = NW * EPW - E
    if pad:
        src = jnp.concatenate([src, jnp.zeros((pad,), src.dtype)])
        dst = jnp.concatenate([dst, jnp.full((pad,), N, dst.dtype)])
    src_r = src.reshape(NW, NCH, CHUNK)
    dst_r = dst.reshape(NW, NCH, CHUNK)
    zero128 = jnp.zeros((CHUNK, C), jnp.float32)
    one128 = jnp.ones((CHUNK, C), jnp.float32)

    y = _tc(_linear_body, 1, x, W)
    degp = _deg(dst_r, zero128, one128)
    dinv, z0 = _tc(_prep_body, 2, degp[:N], degp[AGG_ROWS:AGG_ROWS + N], y)
    a1 = _hop(z0, src_r, dst_r, zero128)
    z1 = _tc(_mid_body, 1, a1[:N], a1[AGG_ROWS:AGG_ROWS + N], z0, dinv)
    a2 = _hop(z1, src_r, dst_r, zero128)
    return _tc(_final_body, 1, a2[:N], a2[AGG_ROWS:AGG_ROWS + N], z1, dinv,
               b.reshape(1, C))


# speedup vs baseline: 54.2762x; 1.2928x over previous
import jax
import jax.numpy as jnp
from jax import lax
from jax.experimental import pallas as pl
from jax.experimental.pallas import tpu as pltpu
from jax.experimental.pallas import tpu_sc as plsc

N = 10000
E = 320000
D = 128
C = 16
NC = 2
NS = 16
NW = NC * NS
CHUNK = 128
EPW = -(-(E // NW) // CHUNK) * CHUNK
NCH = EPW // CHUNK
AGG_ROWS = 10240
RPS = AGG_ROWS // NS


_mesh = plsc.VectorSubcoreMesh(core_axis_name="c", subcore_axis_name="s")
_sc_params = pltpu.CompilerParams(use_tc_tiling_on_sc=False)


NBUF = 8
NG = 3


def _hop_body(tab, src_r, dst_r, zero128, out, src_v, dst_v, rows_v, gsem,
              ssem, agg, ztab):
    c = lax.axis_index("c")
    s = lax.axis_index("s")
    wid = c * NS + s
    pltpu.sync_copy(tab.at[pl.ds(s * (N // NS), N // NS)],
                    ztab.at[pl.ds(s * (N // NS), N // NS)])
    pltpu.sync_copy(zero128, rows_v.at[0])
    for k in range(RPS // CHUNK):
        pltpu.sync_copy(rows_v.at[0], agg.at[pl.ds(s * RPS + k * CHUNK, CHUNK)])
    plsc.subcore_barrier()
    pltpu.sync_copy(src_r.at[wid], src_v)
    pltpu.sync_copy(dst_r.at[wid], dst_v)

    for j in range(NG):
        pltpu.async_copy(ztab.at[src_v.at[j]], rows_v.at[j], gsem)

    def chunk(j, carry):
        @pl.when(j >= NBUF - NG)
        def _():
            pltpu.make_async_copy(zero128, rows_v.at[0], ssem).wait()

        @pl.when(j + NG < NCH)
        def _():
            pltpu.async_copy(ztab.at[src_v.at[lax.rem(j + NG, NCH)]],
                             rows_v.at[lax.rem(j + NG, NBUF)], gsem)

        pltpu.make_async_copy(zero128, rows_v.at[0], gsem).wait()
        pltpu.async_copy(rows_v.at[lax.rem(j, NBUF)], agg.at[dst_v.at[j]],
                         ssem, add=True)
        return carry

    lax.fori_loop(0, NCH, chunk, 0)

    def drain(j, carry):
        pltpu.make_async_copy(zero128, rows_v.at[0], ssem).wait()
        return carry

    lax.fori_loop(0, NBUF - NG, drain, 0)
    plsc.subcore_barrier()
    pltpu.sync_copy(agg.at[pl.ds(s * RPS, RPS)],
                    out.at[pl.ds(c * AGG_ROWS + s * RPS, RPS)])


_hop = pl.kernel(
    _hop_body,
    out_type=jax.ShapeDtypeStruct((NC * AGG_ROWS, C), jnp.float32),
    mesh=_mesh,
    scratch_types=[
        pltpu.VMEM((NCH, CHUNK), jnp.int32),
        pltpu.VMEM((NCH, CHUNK), jnp.int32),
        pltpu.VMEM((NBUF, CHUNK, C), jnp.float32),
        pltpu.SemaphoreType.DMA,
        pltpu.SemaphoreType.DMA,
        pltpu.VMEM_SHARED((AGG_ROWS, C), jnp.float32),
        pltpu.VMEM_SHARED((N, C), jnp.float32),
    ],
    compiler_params=_sc_params,
)


def _deg_body(dst_r, zero128, one128, out, dst_v, rows_v, ssem, agg):
    c = lax.axis_index("c")
    s = lax.axis_index("s")
    wid = c * NS + s
    pltpu.sync_copy(zero128, rows_v)
    for k in range(RPS // CHUNK):
        pltpu.sync_copy(rows_v, agg.at[pl.ds(s * RPS + k * CHUNK, CHUNK)])
    plsc.subcore_barrier()
    pltpu.sync_copy(one128, rows_v)
    pltpu.sync_copy(dst_r.at[wid], dst_v)

    def chunk(j, carry):
        pltpu.async_copy(rows_v, agg.at[dst_v.at[j]], ssem, add=True)

        @pl.when(j >= 8)
        def _():
            pltpu.make_async_copy(zero128, rows_v, ssem).wait()

        return carry

    lax.fori_loop(0, NCH, chunk, 0)

    def drain(j, carry):
        pltpu.make_async_copy(zero128, rows_v, ssem).wait()
        return carry

    lax.fori_loop(0, min(NCH, 8), drain, 0)
    plsc.subcore_barrier()
    pltpu.sync_copy(agg.at[pl.ds(s * RPS, RPS)],
                    out.at[pl.ds(c * AGG_ROWS + s * RPS, RPS)])


_deg = pl.kernel(
    _deg_body,
    out_type=jax.ShapeDtypeStruct((NC * AGG_ROWS, C), jnp.float32),
    mesh=_mesh,
    scratch_types=[
        pltpu.VMEM((NCH, CHUNK), jnp.int32),
        pltpu.VMEM((CHUNK, C), jnp.float32),
        pltpu.SemaphoreType.DMA,
        pltpu.VMEM_SHARED((AGG_ROWS, C), jnp.float32),
    ],
    compiler_params=_sc_params,
)


def _linear_body(x_ref, w_ref, o_ref):
    o_ref[...] = lax.dot_general(
        x_ref[...], w_ref[...], (((1,), (1,)), ((), ())),
        preferred_element_type=jnp.float32,
        precision=lax.Precision.HIGHEST,
    )


def _prep_body(da_ref, db_ref, y_ref, dinv_ref, z0_ref):
    dinv = lax.rsqrt(da_ref[...] + db_ref[...] + 1.0)
    dinv_ref[...] = dinv
    z0_ref[...] = dinv * y_ref[...]


def _mid_body(aa_ref, ab_ref, z0_ref, dinv_ref, z1_ref):
    dinv = dinv_ref[...]
    z1_ref[...] = dinv * dinv * (aa_ref[...] + ab_ref[...] + z0_ref[...])


def _final_body(aa_ref, ab_ref, z1_ref, dinv_ref, b_ref, o_ref):
    h2 = dinv_ref[...] * (aa_ref[...] + ab_ref[...] + z1_ref[...])
    logits = h2 + b_ref[...]
    m = jnp.max(logits, axis=1, keepdims=True)
    lse = m + jnp.log(jnp.sum(jnp.exp(logits - m), axis=1, keepdims=True))
    o_ref[...] = logits - lse


def _tc(body, n_out, *ins):
    outs = tuple(jax.ShapeDtypeStruct((N, C), jnp.float32) for _ in range(n_out))
    return pl.pallas_call(body, out_shape=outs if n_out > 1 else outs[0])(*ins)


def kernel(x, edge_index, W, b):
    src, dst = edge_index[0], edge_index[1]
    pad = NW * EPW - E
    if pad:
        src = jnp.concatenate([src, jnp.zeros((pad,), src.dtype)])
        dst = jnp.concatenate([dst, jnp.full((pad,), N, dst.dtype)])
    src_r = src.reshape(NW, NCH, CHUNK)
    dst_r = dst.reshape(NW, NCH, CHUNK)
    zero128 = jnp.zeros((CHUNK, C), jnp.float32)
    one128 = jnp.ones((CHUNK, C), jnp.float32)

    y = _tc(_linear_body, 1, x, W)
    degp = _deg(dst_r, zero128, one128)
    dinv, z0 = _tc(_prep_body, 2, degp[:N], degp[AGG_ROWS:AGG_ROWS + N], y)
    a1 = _hop(z0, src_r, dst_r, zero128)
    z1 = _tc(_mid_body, 1, a1[:N], a1[AGG_ROWS:AGG_ROWS + N], z0, dinv)
    a2 = _hop(z1, src_r, dst_r, zero128)
    return _tc(_final_body, 1, a2[:N], a2[AGG_ROWS:AGG_ROWS + N], z1, dinv,
               b.reshape(1, C))


# speedup vs baseline: 69.2066x; 1.2751x over previous
import jax
import jax.numpy as jnp
from jax import lax
from jax.experimental import pallas as pl
from jax.experimental.pallas import tpu as pltpu
from jax.experimental.pallas import tpu_sc as plsc

N = 10000
E = 320000
D = 128
C = 16
NC = 2
NS = 16
NW = NC * NS
NPS = N // NS
CHUNK = 128
EPW = -(-(E // NW) // CHUNK) * CHUNK
NCH = EPW // CHUNK
AGG_ROWS = 10240
RPS = AGG_ROWS // NS
NBUF = 8
NG = 3

_mesh = plsc.VectorSubcoreMesh(core_axis_name="c", subcore_axis_name="s")
_sc_params = pltpu.CompilerParams(use_tc_tiling_on_sc=False,
                                  needs_layout_passes=False)


def _rsqrt16(x):
    i = plsc.bitcast(x, jnp.int32)
    g = plsc.bitcast(jnp.int32(0x5F3759DF) - (i >> 1), jnp.float32)
    for _ in range(3):
        g = g * (1.5 - 0.5 * x * g * g)
    return g


def _zero_agg(zero128, rows0, agg, s):
    pltpu.sync_copy(zero128, rows0)
    for k in range(RPS // CHUNK):
        pltpu.sync_copy(rows0, agg.at[pl.ds(s * RPS + k * CHUNK, CHUNK)])


def _scatter_phase(ztab, agg, src_v, dst_v, rows_v, gsem, ssem, zero128):
    for j in range(NG):
        pltpu.async_copy(ztab.at[src_v.at[j]], rows_v.at[j], gsem)

    def chunk(j, carry):
        @pl.when(j >= NBUF - NG)
        def _():
            pltpu.make_async_copy(zero128, rows_v.at[0], ssem).wait()

        @pl.when(j + NG < NCH)
        def _():
            pltpu.async_copy(ztab.at[src_v.at[lax.rem(j + NG, NCH)]],
                             rows_v.at[lax.rem(j + NG, NBUF)], gsem)

        pltpu.make_async_copy(zero128, rows_v.at[0], gsem).wait()
        pltpu.async_copy(rows_v.at[lax.rem(j, NBUF)], agg.at[dst_v.at[j]],
                         ssem, add=True)
        return carry

    lax.fori_loop(0, NCH, chunk, 0)

    def drain(j, carry):
        pltpu.make_async_copy(zero128, rows_v.at[0], ssem).wait()
        return carry

    lax.fori_loop(0, NBUF - NG, drain, 0)


def _deg_body(dst_r, zero128, one128, out, dst_v, rows_v, ssem, agg):
    c = lax.axis_index("c")
    s = lax.axis_index("s")
    wid = c * NS + s
    _zero_agg(zero128, rows_v, agg, s)
    plsc.subcore_barrier()
    pltpu.sync_copy(one128, rows_v)
    pltpu.sync_copy(dst_r.at[wid], dst_v)

    def chunk(j, carry):
        pltpu.async_copy(rows_v, agg.at[dst_v.at[j]], ssem, add=True)

        @pl.when(j >= 8)
        def _():
            pltpu.make_async_copy(zero128, rows_v, ssem).wait()

        return carry

    lax.fori_loop(0, NCH, chunk, 0)

    def drain(j, carry):
        pltpu.make_async_copy(zero128, rows_v, ssem).wait()
        return carry

    lax.fori_loop(0, min(NCH, 8), drain, 0)
    plsc.subcore_barrier()
    pltpu.sync_copy(agg.at[pl.ds(s * RPS, RPS)],
                    out.at[pl.ds(c * AGG_ROWS + s * RPS, RPS)])


_deg = pl.kernel(
    _deg_body,
    out_type=jax.ShapeDtypeStruct((NC * AGG_ROWS, C), jnp.float32),
    mesh=_mesh,
    scratch_types=[
        pltpu.VMEM((NCH, CHUNK), jnp.int32),
        pltpu.VMEM((CHUNK, C), jnp.float32),
        pltpu.SemaphoreType.DMA,
        pltpu.VMEM_SHARED((AGG_ROWS, C), jnp.float32),
    ],
    compiler_params=_sc_params,
)


def _hop1_body(y, degp, src_r, dst_r, zero128,
               agg_out, z0_out, dinv_out,
               src_v, dst_v, rows_v, yv, da, db, zv, dv, gsem, ssem,
               agg, ztab):
    c = lax.axis_index("c")
    s = lax.axis_index("s")
    wid = c * NS + s
    r0 = s * NPS
    pltpu.sync_copy(y.at[pl.ds(r0, NPS)], yv)
    pltpu.sync_copy(degp.at[pl.ds(r0, NPS)], da)
    pltpu.sync_copy(degp.at[pl.ds(AGG_ROWS + r0, NPS)], db)

    @plsc.parallel_loop(0, NPS, unroll=4)
    def _(r):
        g = _rsqrt16(da[r] + db[r] + 1.0)
        dv[r] = g
        zv[r] = g * yv[r]

    pltpu.sync_copy(zv, ztab.at[pl.ds(r0, NPS)])

    @pl.when(c == 0)
    def _():
        pltpu.sync_copy(zv, z0_out.at[pl.ds(r0, NPS)])
        pltpu.sync_copy(dv, dinv_out.at[pl.ds(r0, NPS)])

    _zero_agg(zero128, rows_v.at[0], agg, s)
    plsc.subcore_barrier()
    pltpu.sync_copy(src_r.at[wid], src_v)
    pltpu.sync_copy(dst_r.at[wid], dst_v)
    _scatter_phase(ztab, agg, src_v, dst_v, rows_v, gsem, ssem, zero128)
    plsc.subcore_barrier()
    pltpu.sync_copy(agg.at[pl.ds(s * RPS, RPS)],
                    agg_out.at[pl.ds(c * AGG_ROWS + s * RPS, RPS)])


_hop1 = pl.kernel(
    _hop1_body,
    out_type=(jax.ShapeDtypeStruct((NC * AGG_ROWS, C), jnp.float32),
              jax.ShapeDtypeStruct((N, C), jnp.float32),
              jax.ShapeDtypeStruct((N, C), jnp.float32)),
    mesh=_mesh,
    scratch_types=[
        pltpu.VMEM((NCH, CHUNK), jnp.int32),
        pltpu.VMEM((NCH, CHUNK), jnp.int32),
        pltpu.VMEM((NBUF, CHUNK, C), jnp.float32),
        pltpu.VMEM((NPS, C), jnp.float32),
        pltpu.VMEM((NPS, C), jnp.float32),
        pltpu.VMEM((NPS, C), jnp.float32),
        pltpu.VMEM((NPS, C), jnp.float32),
        pltpu.VMEM((NPS, C), jnp.float32),
        pltpu.SemaphoreType.DMA,
        pltpu.SemaphoreType.DMA,
        pltpu.VMEM_SHARED((AGG_ROWS, C), jnp.float32),
        pltpu.VMEM_SHARED((N, C), jnp.float32),
    ],
    compiler_params=_sc_params,
)


def _hop2_body(z0, dinv, agg1, src_r, dst_r, zero128,
               agg_out, z1_out,
               src_v, dst_v, rows_v, z0v, dv, aa, ab, gsem, ssem,
               agg, ztab):
    c = lax.axis_index("c")
    s = lax.axis_index("s")
    wid = c * NS + s
    r0 = s * NPS
    pltpu.sync_copy(z0.at[pl.ds(r0, NPS)], z0v)
    pltpu.sync_copy(dinv.at[pl.ds(r0, NPS)], dv)
    pltpu.sync_copy(agg1.at[pl.ds(r0, NPS)], aa)
    pltpu.sync_copy(agg1.at[pl.ds(AGG_ROWS + r0, NPS)], ab)

    @plsc.parallel_loop(0, NPS, unroll=4)
    def _(r):
        g = dv[r]
        z0v[r] = g * g * (aa[r] + ab[r] + z0v[r])

    pltpu.sync_copy(z0v, ztab.at[pl.ds(r0, NPS)])

    @pl.when(c == 0)
    def _():
        pltpu.sync_copy(z0v, z1_out.at[pl.ds(r0, NPS)])

    _zero_agg(zero128, rows_v.at[0], agg, s)
    plsc.subcore_barrier()
    pltpu.sync_copy(src_r.at[wid], src_v)
    pltpu.sync_copy(dst_r.at[wid], dst_v)
    _scatter_phase(ztab, agg, src_v, dst_v, rows_v, gsem, ssem, zero128)
    plsc.subcore_barrier()
    pltpu.sync_copy(agg.at[pl.ds(s * RPS, RPS)],
                    agg_out.at[pl.ds(c * AGG_ROWS + s * RPS, RPS)])


_hop2 = pl.kernel(
    _hop2_body,
    out_type=(jax.ShapeDtypeStruct((NC * AGG_ROWS, C), jnp.float32),
              jax.ShapeDtypeStruct((N, C), jnp.float32)),
    mesh=_mesh,
    scratch_types=[
        pltpu.VMEM((NCH, CHUNK), jnp.int32),
        pltpu.VMEM((NCH, CHUNK), jnp.int32),
        pltpu.VMEM((NBUF, CHUNK, C), jnp.float32),
        pltpu.VMEM((NPS, C), jnp.float32),
        pltpu.VMEM((NPS, C), jnp.float32),
        pltpu.VMEM((NPS, C), jnp.float32),
        pltpu.VMEM((NPS, C), jnp.float32),
        pltpu.SemaphoreType.DMA,
        pltpu.SemaphoreType.DMA,
        pltpu.VMEM_SHARED((AGG_ROWS, C), jnp.float32),
        pltpu.VMEM_SHARED((N, C), jnp.float32),
    ],
    compiler_params=_sc_params,
)


def _linear_body(x_ref, w_ref, o_ref):
    o_ref[...] = lax.dot_general(
        x_ref[...], w_ref[...], (((1,), (1,)), ((), ())),
        preferred_element_type=jnp.float32,
        precision=lax.Precision.HIGHEST,
    )


def _final_body(aa_ref, ab_ref, z1_ref, dinv_ref, b_ref, o_ref):
    h2 = dinv_ref[...] * (aa_ref[...] + ab_ref[...] + z1_ref[...])
    logits = h2 + b_ref[...]
    m = jnp.max(logits, axis=1, keepdims=True)
    lse = m + jnp.log(jnp.sum(jnp.exp(logits - m), axis=1, keepdims=True))
    o_ref[...] = logits - lse


def kernel(x, edge_index, W, b):
    src, dst = edge_index[0], edge_index[1]
    pad = NW * EPW - E
    if pad:
        src = jnp.concatenate([src, jnp.zeros((pad,), src.dtype)])
        dst = jnp.concatenate([dst, jnp.full((pad,), N, dst.dtype)])
    src_r = src.reshape(NW, NCH, CHUNK)
    dst_r = dst.reshape(NW, NCH, CHUNK)
    zero128 = jnp.zeros((CHUNK, C), jnp.float32)
    one128 = jnp.ones((CHUNK, C), jnp.float32)

    y = pl.pallas_call(
        _linear_body, out_shape=jax.ShapeDtypeStruct((N, C), jnp.float32),
    )(x, W)
    degp = _deg(dst_r, zero128, one128)
    agg1, z0, dinv = _hop1(y, degp, src_r, dst_r, zero128)
    agg2, z1 = _hop2(z0, dinv, agg1, src_r, dst_r, zero128)
    return pl.pallas_call(
        _final_body, out_shape=jax.ShapeDtypeStruct((N, C), jnp.float32),
    )(agg2[:N], agg2[AGG_ROWS:AGG_ROWS + N], z1, dinv, b.reshape(1, C))


# speedup vs baseline: 75.7210x; 1.0941x over previous
import jax
import jax.numpy as jnp
from jax import lax
from jax.experimental import pallas as pl
from jax.experimental.pallas import tpu as pltpu
from jax.experimental.pallas import tpu_sc as plsc

N = 10000
E = 320000
D = 128
C = 16
NC = 2
NS = 16
NW = NC * NS
NPS = N // NS
CHUNK = 128
ECH = E // CHUNK
NCH = -(-ECH // NW)
NXW = ECH - (NCH - 1) * NW
AGG_ROWS = 10240
RPS = AGG_ROWS // NS
NBUF = 8
NG = 3

_mesh = plsc.VectorSubcoreMesh(core_axis_name="c", subcore_axis_name="s")
_sc_params = pltpu.CompilerParams(use_tc_tiling_on_sc=False,
                                  needs_layout_passes=False)


def _rsqrt16(x):
    i = plsc.bitcast(x, jnp.int32)
    g = plsc.bitcast(jnp.int32(0x5F3759DF) - (i >> 1), jnp.float32)
    for _ in range(3):
        g = g * (1.5 - 0.5 * x * g * g)
    return g


def _zero_agg(zero128, rows0, agg, s):
    pltpu.sync_copy(zero128, rows0)
    for k in range(RPS // CHUNK):
        pltpu.sync_copy(rows0, agg.at[pl.ds(s * RPS + k * CHUNK, CHUNK)])


def _stage_idx(idx_m, idx_v, wid):
    start = jnp.where(wid < NXW, NCH * wid,
                      NXW * NCH + (NCH - 1) * (wid - NXW))

    @pl.when(wid < NXW)
    def _():
        pltpu.sync_copy(idx_m.at[pl.ds(start, NCH)], idx_v)

    @pl.when(wid >= NXW)
    def _():
        pltpu.sync_copy(idx_m.at[pl.ds(start, NCH - 1)],
                        idx_v.at[pl.ds(0, NCH - 1)])


def _scatter_phase(ztab, agg, src_v, dst_v, rows_v, gsem, ssem, zero128, nch):
    for j in range(NG):
        pltpu.async_copy(ztab.at[src_v.at[j]], rows_v.at[j], gsem)

    def chunk(j, carry):
        @pl.when(j >= NBUF - NG)
        def _():
            pltpu.make_async_copy(zero128, rows_v.at[0], ssem).wait()

        @pl.when(j + NG < nch)
        def _():
            pltpu.async_copy(ztab.at[src_v.at[lax.rem(j + NG, NCH)]],
                             rows_v.at[lax.rem(j + NG, NBUF)], gsem)

        pltpu.make_async_copy(zero128, rows_v.at[0], gsem).wait()
        pltpu.async_copy(rows_v.at[lax.rem(j, NBUF)], agg.at[dst_v.at[j]],
                         ssem, add=True)
        return carry

    lax.fori_loop(0, nch, chunk, 0)

    def drain(j, carry):
        pltpu.make_async_copy(zero128, rows_v.at[0], ssem).wait()
        return carry

    lax.fori_loop(0, NBUF - NG, drain, 0)


def _deg_body(dst_m, zero128, one128, out, dst_v, rows_v, ssem, agg):
    c = lax.axis_index("c")
    s = lax.axis_index("s")
    wid = c * NS + s
    nch = jnp.where(wid < NXW, NCH, NCH - 1)
    _zero_agg(zero128, rows_v, agg, s)
    plsc.subcore_barrier()
    pltpu.sync_copy(one128, rows_v)
    _stage_idx(dst_m, dst_v, wid)

    def chunk(j, carry):
        pltpu.async_copy(rows_v, agg.at[dst_v.at[j]], ssem, add=True)

        @pl.when(j >= 8)
        def _():
            pltpu.make_async_copy(zero128, rows_v, ssem).wait()

        return carry

    lax.fori_loop(0, nch, chunk, 0)

    def drain(j, carry):
        pltpu.make_async_copy(zero128, rows_v, ssem).wait()
        return carry

    lax.fori_loop(0, 8, drain, 0)
    plsc.subcore_barrier()
    pltpu.sync_copy(agg.at[pl.ds(s * RPS, RPS)],
                    out.at[pl.ds(c * AGG_ROWS + s * RPS, RPS)])


_deg = pl.kernel(
    _deg_body,
    out_type=jax.ShapeDtypeStruct((NC * AGG_ROWS, C), jnp.float32),
    mesh=_mesh,
    scratch_types=[
        pltpu.VMEM((NCH, CHUNK), jnp.int32),
        pltpu.VMEM((CHUNK, C), jnp.float32),
        pltpu.SemaphoreType.DMA,
        pltpu.VMEM_SHARED((AGG_ROWS, C), jnp.float32),
    ],
    compiler_params=_sc_params,
)


def _hop1_body(y, degp, src_m, dst_m, zero128,
               agg_out, z0_out, dinv_out,
               src_v, dst_v, rows_v, yv, da, db, zv, dv, gsem, ssem,
               agg, ztab):
    c = lax.axis_index("c")
    s = lax.axis_index("s")
    wid = c * NS + s
    r0 = s * NPS
    pltpu.sync_copy(y.at[pl.ds(r0, NPS)], yv)
    pltpu.sync_copy(degp.at[pl.ds(r0, NPS)], da)
    pltpu.sync_copy(degp.at[pl.ds(AGG_ROWS + r0, NPS)], db)

    @plsc.parallel_loop(0, NPS, unroll=4)
    def _(r):
        g = _rsqrt16(da[r] + db[r] + 1.0)
        dv[r] = g
        zv[r] = g * yv[r]

    pltpu.sync_copy(zv, ztab.at[pl.ds(r0, NPS)])

    @pl.when(c == 0)
    def _():
        pltpu.sync_copy(zv, z0_out.at[pl.ds(r0, NPS)])
        pltpu.sync_copy(dv, dinv_out.at[pl.ds(r0, NPS)])

    _zero_agg(zero128, rows_v.at[0], agg, s)
    nch = jnp.where(wid < NXW, NCH, NCH - 1)
    _stage_idx(src_m, src_v, wid)
    _stage_idx(dst_m, dst_v, wid)
    plsc.subcore_barrier()
    _scatter_phase(ztab, agg, src_v, dst_v, rows_v, gsem, ssem, zero128, nch)
    plsc.subcore_barrier()
    pltpu.sync_copy(agg.at[pl.ds(s * RPS, RPS)],
                    agg_out.at[pl.ds(c * AGG_ROWS + s * RPS, RPS)])


_hop1 = pl.kernel(
    _hop1_body,
    out_type=(jax.ShapeDtypeStruct((NC * AGG_ROWS, C), jnp.float32),
              jax.ShapeDtypeStruct((N, C), jnp.float32),
              jax.ShapeDtypeStruct((N, C), jnp.float32)),
    mesh=_mesh,
    scratch_types=[
        pltpu.VMEM((NCH, CHUNK), jnp.int32),
        pltpu.VMEM((NCH, CHUNK), jnp.int32),
        pltpu.VMEM((NBUF, CHUNK, C), jnp.float32),
        pltpu.VMEM((NPS, C), jnp.float32),
        pltpu.VMEM((NPS, C), jnp.float32),
        pltpu.VMEM((NPS, C), jnp.float32),
        pltpu.VMEM((NPS, C), jnp.float32),
        pltpu.VMEM((NPS, C), jnp.float32),
        pltpu.SemaphoreType.DMA,
        pltpu.SemaphoreType.DMA,
        pltpu.VMEM_SHARED((AGG_ROWS, C), jnp.float32),
        pltpu.VMEM_SHARED((N, C), jnp.float32),
    ],
    compiler_params=_sc_params,
)


def _hop2_body(z0, dinv, agg1, src_m, dst_m, zero128,
               agg_out, z1_out,
               src_v, dst_v, rows_v, z0v, dv, aa, ab, gsem, ssem,
               agg, ztab):
    c = lax.axis_index("c")
    s = lax.axis_index("s")
    wid = c * NS + s
    r0 = s * NPS
    pltpu.sync_copy(z0.at[pl.ds(r0, NPS)], z0v)
    pltpu.sync_copy(dinv.at[pl.ds(r0, NPS)], dv)
    pltpu.sync_copy(agg1.at[pl.ds(r0, NPS)], aa)
    pltpu.sync_copy(agg1.at[pl.ds(AGG_ROWS + r0, NPS)], ab)

    @plsc.parallel_loop(0, NPS, unroll=4)
    def _(r):
        g = dv[r]
        z0v[r] = g * g * (aa[r] + ab[r] + z0v[r])

    pltpu.sync_copy(z0v, ztab.at[pl.ds(r0, NPS)])

    @pl.when(c == 0)
    def _():
        pltpu.sync_copy(z0v, z1_out.at[pl.ds(r0, NPS)])

    _zero_agg(zero128, rows_v.at[0], agg, s)
    nch = jnp.where(wid < NXW, NCH, NCH - 1)
    _stage_idx(src_m, src_v, wid)
    _stage_idx(dst_m, dst_v, wid)
    plsc.subcore_barrier()
    _scatter_phase(ztab, agg, src_v, dst_v, rows_v, gsem, ssem, zero128, nch)
    plsc.subcore_barrier()
    pltpu.sync_copy(agg.at[pl.ds(s * RPS, RPS)],
                    agg_out.at[pl.ds(c * AGG_ROWS + s * RPS, RPS)])


_hop2 = pl.kernel(
    _hop2_body,
    out_type=(jax.ShapeDtypeStruct((NC * AGG_ROWS, C), jnp.float32),
              jax.ShapeDtypeStruct((N, C), jnp.float32)),
    mesh=_mesh,
    scratch_types=[
        pltpu.VMEM((NCH, CHUNK), jnp.int32),
        pltpu.VMEM((NCH, CHUNK), jnp.int32),
        pltpu.VMEM((NBUF, CHUNK, C), jnp.float32),
        pltpu.VMEM((NPS, C), jnp.float32),
        pltpu.VMEM((NPS, C), jnp.float32),
        pltpu.VMEM((NPS, C), jnp.float32),
        pltpu.VMEM((NPS, C), jnp.float32),
        pltpu.SemaphoreType.DMA,
        pltpu.SemaphoreType.DMA,
        pltpu.VMEM_SHARED((AGG_ROWS, C), jnp.float32),
        pltpu.VMEM_SHARED((N, C), jnp.float32),
    ],
    compiler_params=_sc_params,
)


def _linear_body(x_ref, w_ref, o_ref):
    o_ref[...] = lax.dot_general(
        x_ref[...], w_ref[...], (((1,), (1,)), ((), ())),
        preferred_element_type=jnp.float32,
        precision=lax.Precision.HIGHEST,
    )


def _final_body(agg2_ref, z1_ref, dinv_ref, b_ref, o_ref):
    aa = agg2_ref[pl.ds(0, N), :]
    ab = agg2_ref[pl.ds(AGG_ROWS, N), :]
    h2 = dinv_ref[...] * (aa + ab + z1_ref[...])
    logits = h2 + b_ref[...]
    m = jnp.max(logits, axis=1, keepdims=True)
    lse = m + jnp.log(jnp.sum(jnp.exp(logits - m), axis=1, keepdims=True))
    o_ref[...] = logits - lse


def kernel(x, edge_index, W, b):
    src_m = edge_index[0].reshape(ECH, CHUNK)
    dst_m = edge_index[1].reshape(ECH, CHUNK)
    zero128 = jnp.zeros((CHUNK, C), jnp.float32)
    one128 = jnp.ones((CHUNK, C), jnp.float32)

    y = pl.pallas_call(
        _linear_body, out_shape=jax.ShapeDtypeStruct((N, C), jnp.float32),
    )(x, W)
    degp = _deg(dst_m, zero128, one128)
    agg1, z0, dinv = _hop1(y, degp, src_m, dst_m, zero128)
    agg2, z1 = _hop2(z0, dinv, agg1, src_m, dst_m, zero128)
    return pl.pallas_call(
        _final_body, out_shape=jax.ShapeDtypeStruct((N, C), jnp.float32),
    )(agg2, z1, dinv, b.reshape(1, C))


# speedup vs baseline: 80.4003x; 1.0618x over previous
import jax
import jax.numpy as jnp
from jax import lax
from jax.experimental import pallas as pl
from jax.experimental.pallas import tpu as pltpu
from jax.experimental.pallas import tpu_sc as plsc

N = 10000
E = 320000
D = 128
C = 16
NC = 2
NS = 16
NW = NC * NS
NPS = N // NS
CHUNK = 128
ECH = E // CHUNK
NCH = -(-ECH // NW)
NXW = ECH - (NCH - 1) * NW
AGG_ROWS = 10240
RPS = AGG_ROWS // NS
NBUF = 8
NG = 3
NPAD = 10016
NPF = NPAD // NW

_mesh = plsc.VectorSubcoreMesh(core_axis_name="c", subcore_axis_name="s")
_sc_params = pltpu.CompilerParams(use_tc_tiling_on_sc=False,
                                  needs_layout_passes=False)


def _rsqrt16(x):
    i = plsc.bitcast(x, jnp.int32)
    g = plsc.bitcast(jnp.int32(0x5F3759DF) - (i >> 1), jnp.float32)
    for _ in range(3):
        g = g * (1.5 - 0.5 * x * g * g)
    return g


def _zero_agg(zero128, rows0, agg, s):
    pltpu.sync_copy(zero128, rows0)
    for k in range(RPS // CHUNK):
        pltpu.sync_copy(rows0, agg.at[pl.ds(s * RPS + k * CHUNK, CHUNK)])


def _stage_idx(idx_m, idx_v, wid):
    start = jnp.where(wid < NXW, NCH * wid,
                      NXW * NCH + (NCH - 1) * (wid - NXW))

    @pl.when(wid < NXW)
    def _():
        pltpu.sync_copy(idx_m.at[pl.ds(start, NCH)], idx_v)

    @pl.when(wid >= NXW)
    def _():
        pltpu.sync_copy(idx_m.at[pl.ds(start, NCH - 1)],
                        idx_v.at[pl.ds(0, NCH - 1)])


def _scatter_phase(ztab, agg, src_v, dst_v, rows_v, gsem, ssem, zero128, nch):
    for j in range(NG):
        pltpu.async_copy(ztab.at[src_v.at[j]], rows_v.at[j], gsem)

    def chunk(j, carry):
        @pl.when(j >= NBUF - NG)
        def _():
            pltpu.make_async_copy(zero128, rows_v.at[0], ssem).wait()

        @pl.when(j + NG < nch)
        def _():
            pltpu.async_copy(ztab.at[src_v.at[lax.rem(j + NG, NCH)]],
                             rows_v.at[lax.rem(j + NG, NBUF)], gsem)

        pltpu.make_async_copy(zero128, rows_v.at[0], gsem).wait()
        pltpu.async_copy(rows_v.at[lax.rem(j, NBUF)], agg.at[dst_v.at[j]],
                         ssem, add=True)
        return carry

    lax.fori_loop(0, nch, chunk, 0)

    def drain(j, carry):
        pltpu.make_async_copy(zero128, rows_v.at[0], ssem).wait()
        return carry

    lax.fori_loop(0, NBUF - NG, drain, 0)


def _deg_body(dst_m, zero128, one128, out, dst_v, rows_v, ssem, agg):
    c = lax.axis_index("c")
    s = lax.axis_index("s")
    wid = c * NS + s
    nch = jnp.where(wid < NXW, NCH, NCH - 1)
    _zero_agg(zero128, rows_v, agg, s)
    plsc.subcore_barrier()
    pltpu.sync_copy(one128, rows_v)
    _stage_idx(dst_m, dst_v, wid)

    def chunk(j, carry):
        pltpu.async_copy(rows_v, agg.at[dst_v.at[j]], ssem, add=True)

        @pl.when(j >= 8)
        def _():
            pltpu.make_async_copy(zero128, rows_v, ssem).wait()

        return carry

    lax.fori_loop(0, nch, chunk, 0)

    def drain(j, carry):
        pltpu.make_async_copy(zero128, rows_v, ssem).wait()
        return carry

    lax.fori_loop(0, 8, drain, 0)
    plsc.subcore_barrier()
    pltpu.sync_copy(agg.at[pl.ds(s * RPS, RPS)],
                    out.at[pl.ds(c * AGG_ROWS + s * RPS, RPS)])


_deg = pl.kernel(
    _deg_body,
    out_type=jax.ShapeDtypeStruct((NC * AGG_ROWS, C), jnp.float32),
    mesh=_mesh,
    scratch_types=[
        pltpu.VMEM((NCH, CHUNK), jnp.int32),
        pltpu.VMEM((CHUNK, C), jnp.float32),
        pltpu.SemaphoreType.DMA,
        pltpu.VMEM_SHARED((AGG_ROWS, C), jnp.float32),
    ],
    compiler_params=_sc_params,
)


def _hop1_body(y, degp, src_m, dst_m, zero128,
               agg_out, z0_out, dinv_out,
               src_v, dst_v, rows_v, yv, da, db, zv, dv, gsem, ssem,
               agg, ztab):
    c = lax.axis_index("c")
    s = lax.axis_index("s")
    wid = c * NS + s
    r0 = s * NPS
    pltpu.sync_copy(y.at[pl.ds(r0, NPS)], yv)
    pltpu.sync_copy(degp.at[pl.ds(r0, NPS)], da)
    pltpu.sync_copy(degp.at[pl.ds(AGG_ROWS + r0, NPS)], db)

    @plsc.parallel_loop(0, NPS, unroll=4)
    def _(r):
        g = _rsqrt16(da[r] + db[r] + 1.0)
        dv[r] = g
        zv[r] = g * yv[r]

    pltpu.sync_copy(zv, ztab.at[pl.ds(r0, NPS)])

    @pl.when(c == 0)
    def _():
        pltpu.sync_copy(zv, z0_out.at[pl.ds(r0, NPS)])
        pltpu.sync_copy(dv, dinv_out.at[pl.ds(r0, NPS)])

    _zero_agg(zero128, rows_v.at[0], agg, s)
    nch = jnp.where(wid < NXW, NCH, NCH - 1)
    _stage_idx(src_m, src_v, wid)
    _stage_idx(dst_m, dst_v, wid)
    plsc.subcore_barrier()
    _scatter_phase(ztab, agg, src_v, dst_v, rows_v, gsem, ssem, zero128, nch)
    plsc.subcore_barrier()
    pltpu.sync_copy(agg.at[pl.ds(s * RPS, RPS)],
                    agg_out.at[pl.ds(c * AGG_ROWS + s * RPS, RPS)])


_hop1 = pl.kernel(
    _hop1_body,
    out_type=(jax.ShapeDtypeStruct((NC * AGG_ROWS, C), jnp.float32),
              jax.ShapeDtypeStruct((NPAD, C), jnp.float32),
              jax.ShapeDtypeStruct((NPAD, C), jnp.float32)),
    mesh=_mesh,
    scratch_types=[
        pltpu.VMEM((NCH, CHUNK), jnp.int32),
        pltpu.VMEM((NCH, CHUNK), jnp.int32),
        pltpu.VMEM((NBUF, CHUNK, C), jnp.float32),
        pltpu.VMEM((NPS, C), jnp.float32),
        pltpu.VMEM((NPS, C), jnp.float32),
        pltpu.VMEM((NPS, C), jnp.float32),
        pltpu.VMEM((NPS, C), jnp.float32),
        pltpu.VMEM((NPS, C), jnp.float32),
        pltpu.SemaphoreType.DMA,
        pltpu.SemaphoreType.DMA,
        pltpu.VMEM_SHARED((AGG_ROWS, C), jnp.float32),
        pltpu.VMEM_SHARED((N, C), jnp.float32),
    ],
    compiler_params=_sc_params,
)


def _hop2_body(z0, dinv, agg1, src_m, dst_m, zero128,
               agg_out, z1_out,
               src_v, dst_v, rows_v, z0v, dv, aa, ab, gsem, ssem,
               agg, ztab):
    c = lax.axis_index("c")
    s = lax.axis_index("s")
    wid = c * NS + s
    r0 = s * NPS
    pltpu.sync_copy(z0.at[pl.ds(r0, NPS)], z0v)
    pltpu.sync_copy(dinv.at[pl.ds(r0, NPS)], dv)
    pltpu.sync_copy(agg1.at[pl.ds(r0, NPS)], aa)
    pltpu.sync_copy(agg1.at[pl.ds(AGG_ROWS + r0, NPS)], ab)

    @plsc.parallel_loop(0, NPS, unroll=4)
    def _(r):
        g = dv[r]
        z0v[r] = g * g * (aa[r] + ab[r] + z0v[r])

    pltpu.sync_copy(z0v, ztab.at[pl.ds(r0, NPS)])

    @pl.when(c == 0)
    def _():
        pltpu.sync_copy(z0v, z1_out.at[pl.ds(r0, NPS)])

    _zero_agg(zero128, rows_v.at[0], agg, s)
    nch = jnp.where(wid < NXW, NCH, NCH - 1)
    _stage_idx(src_m, src_v, wid)
    _stage_idx(dst_m, dst_v, wid)
    plsc.subcore_barrier()
    _scatter_phase(ztab, agg, src_v, dst_v, rows_v, gsem, ssem, zero128, nch)
    plsc.subcore_barrier()
    pltpu.sync_copy(agg.at[pl.ds(s * RPS, RPS)],
                    agg_out.at[pl.ds(c * AGG_ROWS + s * RPS, RPS)])


_hop2 = pl.kernel(
    _hop2_body,
    out_type=(jax.ShapeDtypeStruct((NC * AGG_ROWS, C), jnp.float32),
              jax.ShapeDtypeStruct((NPAD, C), jnp.float32)),
    mesh=_mesh,
    scratch_types=[
        pltpu.VMEM((NCH, CHUNK), jnp.int32),
        pltpu.VMEM((NCH, CHUNK), jnp.int32),
        pltpu.VMEM((NBUF, CHUNK, C), jnp.float32),
        pltpu.VMEM((NPS, C), jnp.float32),
        pltpu.VMEM((NPS, C), jnp.float32),
        pltpu.VMEM((NPS, C), jnp.float32),
        pltpu.VMEM((NPS, C), jnp.float32),
        pltpu.SemaphoreType.DMA,
        pltpu.SemaphoreType.DMA,
        pltpu.VMEM_SHARED((AGG_ROWS, C), jnp.float32),
        pltpu.VMEM_SHARED((N, C), jnp.float32),
    ],
    compiler_params=_sc_params,
)


def _ln16(x):
    bits = plsc.bitcast(x, jnp.int32)
    ef = (bits >> 23) - 127
    m = plsc.bitcast((bits & jnp.int32(0x007FFFFF)) | jnp.int32(0x3F800000),
                     jnp.float32)
    u = (m - 1.0) / (m + 1.0)
    u2 = u * u
    s = 1.0 + u2 * (1.0 / 3.0 + u2 * (0.2 + u2 * (1.0 / 7.0 + u2 / 9.0)))
    return 0.6931471805599453 * ef.astype(jnp.float32) + 2.0 * u * s


def _fin_body(agg2, z1, dinv, bvec, out, aa, ab, zv, dv, bb, ov):
    c = lax.axis_index("c")
    s = lax.axis_index("s")
    wid = c * NS + s
    r0 = wid * NPF
    pltpu.sync_copy(agg2.at[pl.ds(r0, NPF)], aa)
    pltpu.sync_copy(agg2.at[pl.ds(AGG_ROWS + r0, NPF)], ab)
    pltpu.sync_copy(z1.at[pl.ds(r0, NPF)], zv)
    pltpu.sync_copy(dinv.at[pl.ds(r0, NPF)], dv)
    pltpu.sync_copy(bvec, bb)
    bv = bb[...]

    @plsc.parallel_loop(0, NPF, unroll=4)
    def _(r):
        logits = dv[r] * (aa[r] + ab[r] + zv[r]) + bv
        mx = lax.reduce_max(logits, axes=(0,))
        t = logits - mx
        sm = lax.reduce_sum(jnp.exp(t), axes=(0,))
        lnsm = _ln16(jnp.full((C,), sm, jnp.float32))
        ov[r] = t - lnsm

    pltpu.sync_copy(ov, out.at[pl.ds(r0, NPF)])


_fin = pl.kernel(
    _fin_body,
    out_type=jax.ShapeDtypeStruct((NPAD, C), jnp.float32),
    mesh=_mesh,
    scratch_types=[
        pltpu.VMEM((NPF, C), jnp.float32),
        pltpu.VMEM((NPF, C), jnp.float32),
        pltpu.VMEM((NPF, C), jnp.float32),
        pltpu.VMEM((NPF, C), jnp.float32),
        pltpu.VMEM((C,), jnp.float32),
        pltpu.VMEM((NPF, C), jnp.float32),
    ],
    compiler_params=_sc_params,
)


def _linear_body(x_ref, w_ref, o_ref):
    o_ref[...] = lax.dot_general(
        x_ref[...], w_ref[...], (((1,), (1,)), ((), ())),
        preferred_element_type=jnp.float32,
        precision=lax.Precision.HIGHEST,
    )


def kernel(x, edge_index, W, b):
    src_m = edge_index[0].reshape(ECH, CHUNK)
    dst_m = edge_index[1].reshape(ECH, CHUNK)
    zero128 = jnp.zeros((CHUNK, C), jnp.float32)
    one128 = jnp.ones((CHUNK, C), jnp.float32)

    y = pl.pallas_call(
        _linear_body, out_shape=jax.ShapeDtypeStruct((N, C), jnp.float32),
    )(x, W)
    degp = _deg(dst_m, zero128, one128)
    agg1, z0, dinv = _hop1(y, degp, src_m, dst_m, zero128)
    agg2, z1 = _hop2(z0, dinv, agg1, src_m, dst_m, zero128)
    return _fin(agg2, z1, dinv, b)[:N]
